# baseline (speedup 1.0000x reference)
import functools

import jax
import jax.numpy as jnp
from jax import lax
from jax.experimental import pallas as pl
from jax.experimental.pallas import tpu as pltpu
from jax.experimental.pallas import tpu_sc as plsc

NUM_CORES = 2
LANES = 16
NW = 32

BATCH = 16384
EMBED = 32
B_PER_W = BATCH // NW
NBUF = 16


def _k1_body(ut_hbm, uidx_hbm, uw_hbm, uidx_v, tbuf, ucols, sem):
    wid = lax.axis_index("s") * NUM_CORES + lax.axis_index("c")
    base = wid * B_PER_W

    pltpu.sync_copy(uidx_hbm.at[pl.ds(base, B_PER_W)],
                    uidx_v.at[pl.ds(0, B_PER_W)])
    iota = lax.iota(jnp.int32, LANES)

    def idx_at(e):
        return uidx_v[pl.ds(e, LANES)][0]

    def fire(e, slot):
        c = idx_at(e)
        col0 = pl.multiple_of((c >> 7) * 128, 128)
        pltpu.async_copy(ut_hbm.at[:, pl.ds(col0, 128)], tbuf.at[slot], sem)

    def extract(e, slot):
        c = idx_at(e)
        lane = jnp.full((LANES,), c & 127, jnp.int32)
        slot_v = jnp.zeros((LANES,), jnp.int32) + slot
        g0 = plsc.load_gather(tbuf, [slot_v, iota, lane])
        g1 = plsc.load_gather(tbuf, [slot_v, iota + LANES, lane])
        ucols[pl.ds(e * EMBED, LANES)] = g0
        ucols[pl.ds(e * EMBED + LANES, LANES)] = g1

    for e in range(NBUF):
        fire(e, e)

    def step(e, _):
        slot = lax.rem(e, NBUF)
        pltpu.make_async_copy(ut_hbm.at[:, pl.ds(0, 128)],
                              tbuf.at[slot], sem).wait()
        extract(e, slot)
        fire(e + NBUF, lax.rem(e + NBUF, NBUF))
        return _

    lax.fori_loop(0, B_PER_W - NBUF, step, None)

    def tail(e, _):
        slot = lax.rem(e, NBUF)
        pltpu.make_async_copy(ut_hbm.at[:, pl.ds(0, 128)],
                              tbuf.at[slot], sem).wait()
        extract(e, slot)
        return _

    lax.fori_loop(B_PER_W - NBUF, B_PER_W, tail, None)

    pltpu.sync_copy(ucols, uw_hbm.at[pl.ds(base * EMBED, B_PER_W * EMBED)])


_k1 = functools.partial(
    pl.kernel,
    out_type=jax.ShapeDtypeStruct((BATCH * EMBED,), jnp.float32),
    mesh=plsc.VectorSubcoreMesh(core_axis_name="c", subcore_axis_name="s"),
    compiler_params=pltpu.CompilerParams(needs_layout_passes=False),
    scratch_types=[
        pltpu.VMEM((B_PER_W + LANES,), jnp.int32),
        pltpu.VMEM((NBUF, EMBED, 128), jnp.float32),
        pltpu.VMEM((B_PER_W * EMBED,), jnp.float32),
        pltpu.SemaphoreType.DMA,
    ],
)(_k1_body)


def _k2_body(rt_hbm, ridx_hbm, uw_hbm, wb_hbm, out_hbm,
             ridx_v, r_rows, u_flat, wb_v, out_v, sem):
    wid = lax.axis_index("s") * NUM_CORES + lax.axis_index("c")
    base = wid * B_PER_W

    pltpu.sync_copy(wb_hbm, wb_v)
    pltpu.sync_copy(ridx_hbm.at[pl.ds(base, B_PER_W)], ridx_v)
    pltpu.sync_copy(uw_hbm.at[pl.ds(base * EMBED, B_PER_W * EMBED)], u_flat)

    copies = []
    for j in range(4):
        copies.append(pltpu.async_copy(
            rt_hbm.at[ridx_v.at[pl.ds(j * 128, 128)]],
            r_rows.at[pl.ds(j * 128, 128)], sem))
    for c in copies:
        c.wait()

    wvs = [wb_v[pl.ds(d * LANES, LANES)] for d in range(EMBED)]
    bias_v = wb_v[pl.ds(EMBED * LANES, LANES)]
    iota = lax.iota(jnp.int32, LANES)

    def group(g, _):
        rows = g * LANES + iota
        acc = bias_v
        for d in range(EMBED):
            dsp = jnp.full((LANES,), d, jnp.int32)
            uv = plsc.load_gather(u_flat, [rows * EMBED + dsp])
            rv = plsc.load_gather(r_rows, [rows, dsp])
            acc = acc + uv * rv * wvs[d]
        out_v[pl.ds(g * LANES, LANES)] = acc
        return _

    lax.fori_loop(0, B_PER_W // LANES, group, None)
    pltpu.sync_copy(out_v, out_hbm.at[pl.ds(base, B_PER_W)])


_k2 = functools.partial(
    pl.kernel,
    out_type=jax.ShapeDtypeStruct((BATCH,), jnp.float32),
    mesh=plsc.VectorSubcoreMesh(core_axis_name="c", subcore_axis_name="s"),
    compiler_params=pltpu.CompilerParams(needs_layout_passes=False,
                                         use_tc_tiling_on_sc=False),
    scratch_types=[
        pltpu.VMEM((B_PER_W,), jnp.int32),
        pltpu.VMEM((B_PER_W, EMBED), jnp.float32),
        pltpu.VMEM((B_PER_W * EMBED,), jnp.float32),
        pltpu.VMEM(((EMBED + 1) * LANES,), jnp.float32),
        pltpu.VMEM((B_PER_W,), jnp.float32),
        pltpu.SemaphoreType.DMA,
    ],
)(_k2_body)


@jax.jit
def kernel(user, restaurant, user_table, restaurant_table, fc_w, fc_b):
    wb = jnp.concatenate([
        jnp.broadcast_to(fc_w.reshape(EMBED)[:, None],
                         (EMBED, LANES)).reshape(EMBED * LANES),
        jnp.full((LANES,), fc_b[0], jnp.float32),
    ])
    uw = _k1(user_table.T, user.astype(jnp.int32))
    return _k2(restaurant_table, restaurant.astype(jnp.int32), uw, wb)


# speedup vs baseline: 1.0044x; 1.0044x over previous
import functools

import jax
import jax.numpy as jnp
from jax import lax
from jax.experimental import pallas as pl
from jax.experimental.pallas import tpu as pltpu
from jax.experimental.pallas import tpu_sc as plsc

NUM_CORES = 2
LANES = 16
NW = 32

BATCH = 16384
EMBED = 32
B_PER_W = BATCH // NW
NBUF = 8


def _k1_body(ut_hbm, uidx_hbm, uw_hbm, uidx_v, tbuf, ucols, sem):
    wid = lax.axis_index("s") * NUM_CORES + lax.axis_index("c")
    base = wid * B_PER_W

    pltpu.sync_copy(uidx_hbm.at[pl.ds(base, B_PER_W)],
                    uidx_v.at[pl.ds(0, B_PER_W)])
    iota = lax.iota(jnp.int32, LANES)

    def idx_at(e):
        return uidx_v[pl.ds(e, LANES)][0]

    def fire(e, slot):
        c = idx_at(e)
        col0 = pl.multiple_of((c >> 7) * 128, 128)
        pltpu.async_copy(ut_hbm.at[:, pl.ds(col0, 128)], tbuf.at[slot], sem)

    def extract(e, slot):
        c = idx_at(e)
        lane = jnp.full((LANES,), c & 127, jnp.int32)
        slot_v = jnp.zeros((LANES,), jnp.int32) + slot
        g0 = plsc.load_gather(tbuf, [slot_v, iota, lane])
        g1 = plsc.load_gather(tbuf, [slot_v, iota + LANES, lane])
        ucols[pl.ds(e * EMBED, LANES)] = g0
        ucols[pl.ds(e * EMBED + LANES, LANES)] = g1

    for e in range(NBUF):
        fire(e, e)

    def step(e, _):
        slot = lax.rem(e, NBUF)
        pltpu.make_async_copy(ut_hbm.at[:, pl.ds(0, 128)],
                              tbuf.at[slot], sem).wait()
        extract(e, slot)
        fire(e + NBUF, lax.rem(e + NBUF, NBUF))
        return _

    lax.fori_loop(0, B_PER_W - NBUF, step, None)

    def tail(e, _):
        slot = lax.rem(e, NBUF)
        pltpu.make_async_copy(ut_hbm.at[:, pl.ds(0, 128)],
                              tbuf.at[slot], sem).wait()
        extract(e, slot)
        return _

    lax.fori_loop(B_PER_W - NBUF, B_PER_W, tail, None)

    pltpu.sync_copy(ucols, uw_hbm.at[pl.ds(base * EMBED, B_PER_W * EMBED)])


_k1 = functools.partial(
    pl.kernel,
    out_type=jax.ShapeDtypeStruct((BATCH * EMBED,), jnp.float32),
    mesh=plsc.VectorSubcoreMesh(core_axis_name="c", subcore_axis_name="s"),
    compiler_params=pltpu.CompilerParams(needs_layout_passes=False),
    scratch_types=[
        pltpu.VMEM((B_PER_W + LANES,), jnp.int32),
        pltpu.VMEM((NBUF, EMBED, 128), jnp.float32),
        pltpu.VMEM((B_PER_W * EMBED,), jnp.float32),
        pltpu.SemaphoreType.DMA,
    ],
)(_k1_body)


def _k2_body(rt_hbm, ridx_hbm, uw_hbm, wb_hbm, out_hbm,
             ridx_v, r_rows, u_flat, wb_v, out_v, sem):
    wid = lax.axis_index("s") * NUM_CORES + lax.axis_index("c")
    base = wid * B_PER_W

    pltpu.sync_copy(wb_hbm, wb_v)
    pltpu.sync_copy(ridx_hbm.at[pl.ds(base, B_PER_W)], ridx_v)
    pltpu.sync_copy(uw_hbm.at[pl.ds(base * EMBED, B_PER_W * EMBED)], u_flat)

    copies = []
    for j in range(4):
        copies.append(pltpu.async_copy(
            rt_hbm.at[ridx_v.at[pl.ds(j * 128, 128)]],
            r_rows.at[pl.ds(j * 128, 128)], sem))
    for c in copies:
        c.wait()

    wvs = [wb_v[pl.ds(d * LANES, LANES)] for d in range(EMBED)]
    bias_v = wb_v[pl.ds(EMBED * LANES, LANES)]
    iota = lax.iota(jnp.int32, LANES)

    def group(g, _):
        rows = g * LANES + iota
        acc = bias_v
        for d in range(EMBED):
            dsp = jnp.full((LANES,), d, jnp.int32)
            uv = plsc.load_gather(u_flat, [rows * EMBED + dsp])
            rv = plsc.load_gather(r_rows, [rows, dsp])
            acc = acc + uv * rv * wvs[d]
        out_v[pl.ds(g * LANES, LANES)] = acc
        return _

    lax.fori_loop(0, B_PER_W // LANES, group, None)
    pltpu.sync_copy(out_v, out_hbm.at[pl.ds(base, B_PER_W)])


_k2 = functools.partial(
    pl.kernel,
    out_type=jax.ShapeDtypeStruct((BATCH,), jnp.float32),
    mesh=plsc.VectorSubcoreMesh(core_axis_name="c", subcore_axis_name="s"),
    compiler_params=pltpu.CompilerParams(needs_layout_passes=False,
                                         use_tc_tiling_on_sc=False),
    scratch_types=[
        pltpu.VMEM((B_PER_W,), jnp.int32),
        pltpu.VMEM((B_PER_W, EMBED), jnp.float32),
        pltpu.VMEM((B_PER_W * EMBED,), jnp.float32),
        pltpu.VMEM(((EMBED + 1) * LANES,), jnp.float32),
        pltpu.VMEM((B_PER_W,), jnp.float32),
        pltpu.SemaphoreType.DMA,
    ],
)(_k2_body)


@jax.jit
def kernel(user, restaurant, user_table, restaurant_table, fc_w, fc_b):
    wb = jnp.concatenate([
        jnp.broadcast_to(fc_w.reshape(EMBED)[:, None],
                         (EMBED, LANES)).reshape(EMBED * LANES),
        jnp.full((LANES,), fc_b[0], jnp.float32),
    ])
    uw = _k1(user_table.T, user.astype(jnp.int32))
    return _k2(restaurant_table, restaurant.astype(jnp.int32), uw, wb)
